# Initial kernel scaffold; baseline (speedup 1.0000x reference)
import jax
import jax.numpy as jnp
from jax.experimental import pallas as pl


def kernel(expression_values, gene_table, zero_embedding, eW1, eb1, eW2, eb2, eW3, eb3, mW1, mb1, mW2, mb2, cond_table, pW1, pb1, pW2, pb2, xW1, xb1, xW2, xb2, ctxW1, ctxb1, ctxW2, ctxb2, tf_table, pos_table, prefix_type, gene_type, rms_w, input_ids, condition_ids, padding_mask, non_tf_mask):
    raise NotImplementedError("write your pallas kernel here")


# speedup vs baseline: 1.9554x; 1.9554x over previous
import functools

import jax
import jax.numpy as jnp
from jax import lax
from jax.experimental import pallas as pl
from jax.experimental.pallas import tpu as pltpu
from jax.experimental.pallas import tpu_sc as plsc

C, G, D = 32, 2048, 256
_NC, _NS = 2, 16
_NW = _NC * _NS
_CH = 128


def _silu(x):
    return x * jax.nn.sigmoid(x)


def _sc_gather(gene_table, gene_idx, cond_table, cond_idx):
    B = gene_idx.shape[0]
    CB = cond_idx.shape[0]
    rows_w = B // _NW
    nch = rows_w // _CH

    mesh = plsc.VectorSubcoreMesh(core_axis_name="c", subcore_axis_name="s")

    @functools.partial(
        pl.kernel,
        out_type=(
            jax.ShapeDtypeStruct((B, D), jnp.float32),
            jax.ShapeDtypeStruct((CB, D), jnp.float32),
        ),
        mesh=mesh,
        scratch_types=[
            pltpu.VMEM((_CH,), jnp.int32),
            pltpu.VMEM((_CH, D), jnp.float32),
            pltpu.VMEM((CB,), jnp.int32),
            pltpu.VMEM((CB, D), jnp.float32),
            pltpu.SemaphoreType.DMA,
        ],
    )
    def gather_k(table_h, idx_h, ctab_h, cidx_h, out_h, cout_h,
                 idx_v, rows_v, cidx_v, crows_v, sem):
        wid = lax.axis_index("s") * _NC + lax.axis_index("c")
        base = wid * rows_w

        def chunk(k, carry):
            off = base + k * _CH
            pltpu.sync_copy(idx_h.at[pl.ds(off, _CH)], idx_v)
            pltpu.async_copy(table_h.at[idx_v], rows_v, sem).wait()
            pltpu.sync_copy(rows_v, out_h.at[pl.ds(off, _CH)])
            return carry

        lax.fori_loop(0, nch, chunk, 0)

        @pl.when(wid == 0)
        def _():
            pltpu.sync_copy(cidx_h, cidx_v)
            pltpu.async_copy(ctab_h.at[cidx_v], crows_v, sem).wait()
            pltpu.sync_copy(crows_v, cout_h)

    return gather_k(gene_table, gene_idx, cond_table, cond_idx)


def _cell_body(gene_ref, ev_ref, tf_ref, valid_ref, ce_ref,
               eW1_ref, eb1_ref, eW2_ref, eb2_ref, eW3_ref, eb3_ref,
               mW1_ref, mb1_ref, mW2_ref, mb2_ref,
               tf_tab_ref, gene_type_ref, zero_ref,
               ctxW1_ref, ctxb1_ref, ctxW2_ref, ctxb2_ref,
               pW1_ref, pb1_ref, pW2_ref, pb2_ref,
               xW1_ref, xb1_ref, xW2_ref, xb2_ref,
               prefix_type_ref, rms_ref, pos_ref, out_ref):
    f32 = jnp.float32
    v = ev_ref[0]
    h = _silu(v * eW1_ref[...] + eb1_ref[...])
    h = _silu(jnp.dot(h, eW2_ref[...], preferred_element_type=f32)
              + eb2_ref[...])
    expr = jnp.dot(h, eW3_ref[...], preferred_element_type=f32) + eb3_ref[...]
    expr = jnp.where(v == 0.0, zero_ref[...], expr)
    m = _silu(v * mW1_ref[...] + mb1_ref[...])
    mod = jnp.dot(m, mW2_ref[...], preferred_element_type=f32) + mb2_ref[...]
    scale = jax.nn.sigmoid(mod[:, :D])
    shift = mod[:, D:]
    t0 = tf_tab_ref[0:1, :]
    t1 = tf_tab_ref[1:2, :]
    tfm = tf_ref[0]
    tf_emb = t0 + tfm * (t1 - t0)
    tokens = (gene_ref[0] + expr + tf_emb + gene_type_ref[...]) * scale + shift
    valid = valid_ref[0]
    pooled = jnp.sum(tokens * valid, axis=0, keepdims=True) / jnp.maximum(
        jnp.sum(valid), 1.0)
    ctx = jnp.dot(_silu(jnp.dot(pooled, ctxW1_ref[...],
                                preferred_element_type=f32) + ctxb1_ref[...]),
                  ctxW2_ref[...], preferred_element_type=f32) + ctxb2_ref[...]
    ce = ce_ref[0]
    ptok = jnp.dot(_silu(jnp.dot(ce, pW1_ref[...],
                                 preferred_element_type=f32) + pb1_ref[...]),
                   pW2_ref[...], preferred_element_type=f32) + pb2_ref[...]
    cbias = jnp.dot(_silu(jnp.dot(ce, xW1_ref[...],
                                  preferred_element_type=f32) + xb1_ref[...]),
                    xW2_ref[...], preferred_element_type=f32) + xb2_ref[...]
    prefix_row = ptok + ctx + prefix_type_ref[...] + pos_ref[0:1, :]
    genes = tokens + cbias + pos_ref[1:, :]
    full = jnp.concatenate([prefix_row, genes], axis=0)
    norm = full * lax.rsqrt(
        jnp.mean(full * full, axis=-1, keepdims=True) + 1e-6) * rms_ref[...]
    out_ref[0] = norm


def kernel(expression_values, gene_table, zero_embedding, eW1, eb1, eW2, eb2,
           eW3, eb3, mW1, mb1, mW2, mb2, cond_table, pW1, pb1, pW2, pb2,
           xW1, xb1, xW2, xb2, ctxW1, ctxb1, ctxW2, ctxb2, tf_table,
           pos_table, prefix_type, gene_type, rms_w, input_ids,
           condition_ids, padding_mask, non_tf_mask):
    gene_idx = input_ids.reshape(-1).astype(jnp.int32)
    cond_idx = condition_ids.reshape(-1).astype(jnp.int32)
    gathered, ce_rows = _sc_gather(gene_table, gene_idx, cond_table, cond_idx)
    gathered = gathered.reshape(C, G, D)
    ce3 = ce_rows.reshape(C, 1, 4 * D)

    ev3 = expression_values.reshape(C, G, 1)
    tf3 = non_tf_mask.astype(jnp.float32).reshape(C, G, 1)
    valid3 = (~padding_mask).astype(jnp.float32).reshape(C, G, 1)
    pos = pos_table[: G + 1]

    row = lambda b: b.reshape(1, -1)
    weights = (eW1, row(eb1), eW2, row(eb2), eW3, row(eb3),
               mW1, row(mb1), mW2, row(mb2),
               tf_table, gene_type.reshape(1, D), row(zero_embedding),
               ctxW1, row(ctxb1), ctxW2, row(ctxb2),
               pW1, row(pb1), pW2, row(pb2),
               xW1, row(xb1), xW2, row(xb2),
               prefix_type.reshape(1, D), row(rms_w), pos)

    full = lambda a: pl.BlockSpec(a.shape, lambda c: (0,) * a.ndim)
    in_specs = [
        pl.BlockSpec((1, G, D), lambda c: (c, 0, 0)),
        pl.BlockSpec((1, G, 1), lambda c: (c, 0, 0)),
        pl.BlockSpec((1, G, 1), lambda c: (c, 0, 0)),
        pl.BlockSpec((1, G, 1), lambda c: (c, 0, 0)),
        pl.BlockSpec((1, 1, 4 * D), lambda c: (c, 0, 0)),
    ] + [full(w) for w in weights]

    out = pl.pallas_call(
        _cell_body,
        grid=(C,),
        in_specs=in_specs,
        out_specs=pl.BlockSpec((1, G + 1, D), lambda c: (c, 0, 0)),
        out_shape=jax.ShapeDtypeStruct((C, G + 1, D), jnp.float32),
    )(gathered, ev3, tf3, valid3, ce3, *weights)
    return out
